# baseline (speedup 1.0000x reference)
import functools

import jax
import jax.numpy as jnp
from jax import lax
from jax.experimental import pallas as pl
from jax.experimental.pallas import tpu as pltpu
from jax.experimental.pallas import tpu_sc as plsc

_K = 8192
_D = 32
_N = 8192
_TM = 512
_G = _N // _TM
_COMMIT = 0.25


def _argmin_body(x_ref, s1_ref, e_ref, s2_ref, idx_ref, loss_ref):
    x = x_ref[...]
    e = e_ref[...]
    s1 = s1_ref[...]
    xb = (2.0 * x).astype(jnp.bfloat16)
    m2 = lax.dot_general(xb, e, (((1,), (1,)), ((), ())),
                         preferred_element_type=jnp.float32)
    d2 = (s1 - m2) + s2_ref[...]
    dist = jnp.sqrt(jnp.maximum(d2, 0.0))
    h = _K // 2
    da, db = dist[:, :h], dist[:, h:]
    mva = jnp.min(da, axis=1, keepdims=True)
    mvb = jnp.min(db, axis=1, keepdims=True)
    takeb = mvb < mva.astype(jnp.bfloat16).astype(jnp.float32)
    mv = jnp.where(takeb, mvb, mva)
    dw = jnp.where(takeb, db, da)
    ia = lax.broadcasted_iota(jnp.int32, dw.shape, 1)
    idx = jnp.min(jnp.where(dw == mv, ia, jnp.int32(_K)), axis=1)
    idx = idx + jnp.where(takeb[:, 0], jnp.int32(h), 0)
    idx_ref[0, 0, :] = idx

    @pl.when(pl.program_id(0) == 0)
    def _():
        loss_ref[...] = jnp.zeros((1, 1), jnp.float32)

    loss_ref[...] += jnp.sum(mv * mv, axis=0, keepdims=True)


def _tc_argmin(flat_x, s1, emb, s2):
    return pl.pallas_call(
        _argmin_body,
        grid=(_G,),
        in_specs=[
            pl.BlockSpec((_TM, _D), lambda i: (i, 0)),
            pl.BlockSpec((_TM, 1), lambda i: (i, 0)),
            pl.BlockSpec((_K, _D), lambda i: (0, 0)),
            pl.BlockSpec((1, _K), lambda i: (0, 0)),
        ],
        out_specs=[
            pl.BlockSpec((1, 1, _TM), lambda i: (i, 0, 0)),
            pl.BlockSpec((1, 1), lambda i: (0, 0)),
        ],
        out_shape=[
            jax.ShapeDtypeStruct((_G, 1, _TM), jnp.int32),
            jax.ShapeDtypeStruct((1, 1), jnp.float32),
        ],
    )(flat_x, s1, emb, s2)


def _sc_gather(table, idx):
    info = plsc.get_sparse_core_info()
    nw = info.num_cores * info.num_subcores
    bpw = _N // nw
    mesh = plsc.VectorSubcoreMesh(core_axis_name="c", subcore_axis_name="s")

    @functools.partial(
        pl.kernel, mesh=mesh,
        compiler_params=pltpu.CompilerParams(use_tc_tiling_on_sc=False),
        out_type=jax.ShapeDtypeStruct((_N, _D), jnp.float32),
        scratch_types=[
            pltpu.VMEM((bpw,), jnp.int32),
            pltpu.VMEM((bpw, _D), jnp.float32),
            pltpu.SemaphoreType.DMA,
        ],
    )
    def k(table_hbm, idx_hbm, out_hbm, idx_v, rows_v, sem):
        wid = lax.axis_index("s") * info.num_cores + lax.axis_index("c")
        base = wid * bpw
        pltpu.sync_copy(idx_hbm.at[pl.ds(base, bpw)], idx_v)
        pltpu.async_copy(table_hbm.at[idx_v], rows_v, sem).wait()
        pltpu.sync_copy(rows_v, out_hbm.at[pl.ds(base, bpw)])

    return k(table, idx)


def kernel(x, embeddings):
    flat_x = x.reshape(_N, _D)
    s1 = jnp.sum(flat_x ** 2, axis=1, keepdims=True)
    s2 = jnp.sum(embeddings ** 2, axis=1)[None, :]
    idx3, loss_sum = _tc_argmin(flat_x, s1, embeddings, s2)
    idx = idx3.reshape(_N)
    quantized = _sc_gather(embeddings, idx)
    loss = (1.0 + _COMMIT) * (loss_sum[0, 0] / jnp.float32(_N * _D))
    return quantized.reshape(x.shape), loss


# speedup vs baseline: 1.0076x; 1.0076x over previous
import functools

import jax
import jax.numpy as jnp
from jax import lax
from jax.experimental import pallas as pl
from jax.experimental.pallas import tpu as pltpu
from jax.experimental.pallas import tpu_sc as plsc

_K = 8192
_D = 32
_N = 8192
_TM = 512
_G = _N // _TM
_COMMIT = 0.25


def _argmin_body(x_ref, s1_ref, et_ref, s2_ref, idx_ref, loss_ref):
    x = x_ref[...]
    et = et_ref[...]
    s1 = s1_ref[...]
    xb = (2.0 * x).astype(jnp.bfloat16)
    m2 = lax.dot_general(xb, et, (((1,), (0,)), ((), ())),
                         preferred_element_type=jnp.float32)
    d2 = (s1 - m2) + s2_ref[...]
    dist = jnp.sqrt(jnp.maximum(d2, 0.0))
    h = _K // 2
    da, db = dist[:, :h], dist[:, h:]
    mva = jnp.min(da, axis=1, keepdims=True)
    mvb = jnp.min(db, axis=1, keepdims=True)
    takeb = mvb < mva.astype(jnp.bfloat16).astype(jnp.float32)
    mv = jnp.where(takeb, mvb, mva)
    dw = jnp.where(takeb, db, da)
    ia = lax.broadcasted_iota(jnp.int32, dw.shape, 1)
    idx = jnp.min(jnp.where(dw == mv, ia, jnp.int32(_K)), axis=1)
    idx = idx + jnp.where(takeb[:, 0], jnp.int32(h), 0)
    idx_ref[0, 0, :] = idx

    @pl.when(pl.program_id(0) == 0)
    def _():
        loss_ref[...] = jnp.zeros((1, 1), jnp.float32)

    loss_ref[...] += jnp.sum(mv * mv, axis=0, keepdims=True)


def _tc_argmin(flat_x, s1, emb_t, s2):
    return pl.pallas_call(
        _argmin_body,
        grid=(_G,),
        in_specs=[
            pl.BlockSpec((_TM, _D), lambda i: (i, 0)),
            pl.BlockSpec((_TM, 1), lambda i: (i, 0)),
            pl.BlockSpec((_D, _K), lambda i: (0, 0)),
            pl.BlockSpec((1, _K), lambda i: (0, 0)),
        ],
        out_specs=[
            pl.BlockSpec((1, 1, _TM), lambda i: (i, 0, 0)),
            pl.BlockSpec((1, 1), lambda i: (0, 0)),
        ],
        out_shape=[
            jax.ShapeDtypeStruct((_G, 1, _TM), jnp.int32),
            jax.ShapeDtypeStruct((1, 1), jnp.float32),
        ],
    )(flat_x, s1, emb_t, s2)


def _sc_gather(table, idx):
    info = plsc.get_sparse_core_info()
    nw = info.num_cores * info.num_subcores
    bpw = _N // nw
    mesh = plsc.VectorSubcoreMesh(core_axis_name="c", subcore_axis_name="s")

    @functools.partial(
        pl.kernel, mesh=mesh,
        compiler_params=pltpu.CompilerParams(use_tc_tiling_on_sc=False),
        out_type=jax.ShapeDtypeStruct((_N, _D), jnp.float32),
        scratch_types=[
            pltpu.VMEM((bpw,), jnp.int32),
            pltpu.VMEM((bpw, _D), jnp.float32),
            pltpu.SemaphoreType.DMA,
        ],
    )
    def k(table_hbm, idx_hbm, out_hbm, idx_v, rows_v, sem):
        wid = lax.axis_index("s") * info.num_cores + lax.axis_index("c")
        base = wid * bpw
        pltpu.sync_copy(idx_hbm.at[pl.ds(base, bpw)], idx_v)
        pltpu.async_copy(table_hbm.at[idx_v], rows_v, sem).wait()
        pltpu.sync_copy(rows_v, out_hbm.at[pl.ds(base, bpw)])

    return k(table, idx)


def kernel(x, embeddings):
    flat_x = x.reshape(_N, _D)
    s1 = jnp.sum(flat_x ** 2, axis=1, keepdims=True)
    s2 = jnp.sum(embeddings ** 2, axis=1)[None, :]
    idx3, loss_sum = _tc_argmin(flat_x, s1, embeddings.T, s2)
    idx = idx3.reshape(_N)
    quantized = _sc_gather(embeddings, idx)
    loss = (1.0 + _COMMIT) * (loss_sum[0, 0] / jnp.float32(_N * _D))
    return quantized.reshape(x.shape), loss


# speedup vs baseline: 1.0136x; 1.0059x over previous
import functools

import jax
import jax.numpy as jnp
from jax import lax
from jax.experimental import pallas as pl
from jax.experimental.pallas import tpu as pltpu
from jax.experimental.pallas import tpu_sc as plsc

_K = 8192
_D = 32
_N = 8192
_TM = 512
_G = _N // _TM
_COMMIT = 0.25


def _argmin_body(x_ref, s1_ref, et_ref, s2_ref, idx_ref, loss_ref):
    x = x_ref[...]
    et = et_ref[...]
    s1 = s1_ref[...]
    xb = (2.0 * x).astype(jnp.bfloat16)
    m2 = lax.dot_general(xb, et, (((1,), (0,)), ((), ())),
                         preferred_element_type=jnp.float32)
    d2 = (s1 - m2) + s2_ref[...]
    dist = jnp.sqrt(jnp.maximum(d2, 0.0))
    h = _K // 2
    da, db = dist[:, :h], dist[:, h:]
    mva = jnp.min(da, axis=1, keepdims=True)
    mvb = jnp.min(db, axis=1, keepdims=True)
    takeb = mvb < mva.astype(jnp.bfloat16).astype(jnp.float32)
    mv = jnp.where(takeb, mvb, mva)
    dw = jnp.where(takeb, db, da)
    ia = lax.broadcasted_iota(jnp.int32, dw.shape, 1)
    idx = jnp.min(jnp.where(dw == mv, ia, jnp.int32(_K)), axis=1)
    idx = idx + jnp.where(takeb[:, 0], jnp.int32(h), 0)
    idx_ref[0, 0, :] = idx

    @pl.when(pl.program_id(0) == 0)
    def _():
        loss_ref[...] = jnp.zeros((1, 1), jnp.float32)

    loss_ref[...] += jnp.sum(mv * mv, axis=0, keepdims=True)

    @pl.when(pl.program_id(0) == _G - 1)
    def _():
        loss_ref[...] = ((1.0 + _COMMIT) / jnp.float32(_N * _D)) * loss_ref[...]


def _tc_argmin(flat_x, s1, emb_t, s2):
    return pl.pallas_call(
        _argmin_body,
        grid=(_G,),
        in_specs=[
            pl.BlockSpec((_TM, _D), lambda i: (i, 0)),
            pl.BlockSpec((_TM, 1), lambda i: (i, 0)),
            pl.BlockSpec((_D, _K), lambda i: (0, 0)),
            pl.BlockSpec((1, _K), lambda i: (0, 0)),
        ],
        out_specs=[
            pl.BlockSpec((1, 1, _TM), lambda i: (i, 0, 0)),
            pl.BlockSpec((1, 1), lambda i: (0, 0)),
        ],
        out_shape=[
            jax.ShapeDtypeStruct((_G, 1, _TM), jnp.int32),
            jax.ShapeDtypeStruct((1, 1), jnp.float32),
        ],
    )(flat_x, s1, emb_t, s2)


def _sc_gather(table, idx):
    info = plsc.get_sparse_core_info()
    nw = info.num_cores * info.num_subcores
    bpw = _N // nw
    mesh = plsc.VectorSubcoreMesh(core_axis_name="c", subcore_axis_name="s")

    @functools.partial(
        pl.kernel, mesh=mesh,
        compiler_params=pltpu.CompilerParams(use_tc_tiling_on_sc=False),
        out_type=jax.ShapeDtypeStruct((_N, _D), jnp.float32),
        scratch_types=[
            pltpu.VMEM((bpw,), jnp.int32),
            pltpu.VMEM((bpw, _D), jnp.float32),
            pltpu.SemaphoreType.DMA,
        ],
    )
    def k(table_hbm, idx_hbm, out_hbm, idx_v, rows_v, sem):
        wid = lax.axis_index("s") * info.num_cores + lax.axis_index("c")
        base = wid * bpw
        pltpu.sync_copy(idx_hbm.at[pl.ds(base, bpw)], idx_v)
        pltpu.async_copy(table_hbm.at[idx_v], rows_v, sem).wait()
        pltpu.sync_copy(rows_v, out_hbm.at[pl.ds(base, bpw)])

    return k(table, idx)


def kernel(x, embeddings):
    flat_x = x.reshape(_N, _D)
    s1 = jnp.sum(flat_x ** 2, axis=1, keepdims=True)
    s2 = jnp.sum(embeddings ** 2, axis=1)[None, :]
    idx3, loss_sum = _tc_argmin(flat_x, s1, embeddings.T, s2)
    idx = idx3.reshape(_N)
    quantized = _sc_gather(embeddings, idx)
    return quantized.reshape(x.shape), loss_sum.reshape(())


# speedup vs baseline: 1.0971x; 1.0824x over previous
import functools

import jax
import jax.numpy as jnp
from jax import lax
from jax.experimental import pallas as pl
from jax.experimental.pallas import tpu as pltpu
from jax.experimental.pallas import tpu_sc as plsc

_K = 8192
_D = 32
_N = 8192
_TM = 1024
_G = _N // _TM
_COMMIT = 0.25


def _argmin_body(x_ref, s1_ref, et_ref, s2_ref, idx_ref, loss_ref):
    x = x_ref[...]
    et = et_ref[...]
    s1 = s1_ref[...]
    xb = (2.0 * x).astype(jnp.bfloat16)
    m2 = lax.dot_general(xb, et, (((1,), (0,)), ((), ())),
                         preferred_element_type=jnp.float32)
    d2 = (s1 - m2) + s2_ref[...]
    dist = jnp.sqrt(jnp.maximum(d2, 0.0))
    h = _K // 2
    da, db = dist[:, :h], dist[:, h:]
    mva = jnp.min(da, axis=1, keepdims=True)
    mvb = jnp.min(db, axis=1, keepdims=True)
    takeb = mvb < mva.astype(jnp.bfloat16).astype(jnp.float32)
    mv = jnp.where(takeb, mvb, mva)
    dw = jnp.where(takeb, db, da)
    ia = lax.broadcasted_iota(jnp.int32, dw.shape, 1)
    idx = jnp.min(jnp.where(dw == mv, ia, jnp.int32(_K)), axis=1)
    idx = idx + jnp.where(takeb[:, 0], jnp.int32(h), 0)
    idx_ref[0, 0, :] = idx

    @pl.when(pl.program_id(0) == 0)
    def _():
        loss_ref[...] = jnp.zeros((1, 1), jnp.float32)

    loss_ref[...] += jnp.sum(mv * mv, axis=0, keepdims=True)

    @pl.when(pl.program_id(0) == _G - 1)
    def _():
        loss_ref[...] = ((1.0 + _COMMIT) / jnp.float32(_N * _D)) * loss_ref[...]


def _tc_argmin(flat_x, s1, emb_t, s2):
    return pl.pallas_call(
        _argmin_body,
        grid=(_G,),
        in_specs=[
            pl.BlockSpec((_TM, _D), lambda i: (i, 0)),
            pl.BlockSpec((_TM, 1), lambda i: (i, 0)),
            pl.BlockSpec((_D, _K), lambda i: (0, 0)),
            pl.BlockSpec((1, _K), lambda i: (0, 0)),
        ],
        out_specs=[
            pl.BlockSpec((1, 1, _TM), lambda i: (i, 0, 0)),
            pl.BlockSpec((1, 1), lambda i: (0, 0)),
        ],
        out_shape=[
            jax.ShapeDtypeStruct((_G, 1, _TM), jnp.int32),
            jax.ShapeDtypeStruct((1, 1), jnp.float32),
        ],
    )(flat_x, s1, emb_t, s2)


def _sc_gather(table, idx):
    info = plsc.get_sparse_core_info()
    nw = info.num_cores * info.num_subcores
    bpw = _N // nw
    mesh = plsc.VectorSubcoreMesh(core_axis_name="c", subcore_axis_name="s")

    @functools.partial(
        pl.kernel, mesh=mesh,
        compiler_params=pltpu.CompilerParams(use_tc_tiling_on_sc=False),
        out_type=jax.ShapeDtypeStruct((_N, _D), jnp.float32),
        scratch_types=[
            pltpu.VMEM((bpw,), jnp.int32),
            pltpu.VMEM((bpw, _D), jnp.float32),
            pltpu.SemaphoreType.DMA,
        ],
    )
    def k(table_hbm, idx_hbm, out_hbm, idx_v, rows_v, sem):
        wid = lax.axis_index("s") * info.num_cores + lax.axis_index("c")
        base = wid * bpw
        pltpu.sync_copy(idx_hbm.at[pl.ds(base, bpw)], idx_v)
        pltpu.async_copy(table_hbm.at[idx_v], rows_v, sem).wait()
        pltpu.sync_copy(rows_v, out_hbm.at[pl.ds(base, bpw)])

    return k(table, idx)


def kernel(x, embeddings):
    flat_x = x.reshape(_N, _D)
    s1 = jnp.sum(flat_x ** 2, axis=1, keepdims=True)
    s2 = jnp.sum(embeddings ** 2, axis=1)[None, :]
    idx3, loss_sum = _tc_argmin(flat_x, s1, embeddings.T, s2)
    idx = idx3.reshape(_N)
    quantized = _sc_gather(embeddings, idx)
    return quantized.reshape(x.shape), loss_sum.reshape(())


# speedup vs baseline: 1.1722x; 1.0684x over previous
import functools

import jax
import jax.numpy as jnp
from jax import lax
from jax.experimental import pallas as pl
from jax.experimental.pallas import tpu as pltpu
from jax.experimental.pallas import tpu_sc as plsc

_K = 8192
_D = 32
_N = 8192
_TM = 1024
_G = _N // _TM
_COMMIT = 0.25


def _argmin_body(x_ref, s1_ref, et_ref, s2_ref, iota_ref, idx_ref, loss_ref):
    x = x_ref[...]
    et = et_ref[...]
    s1 = s1_ref[...]
    xb = (2.0 * x).astype(jnp.bfloat16)
    m2 = lax.dot_general(xb, et, (((1,), (0,)), ((), ())),
                         preferred_element_type=jnp.float32)
    d2 = (s1 - m2) + s2_ref[...]
    dist = jnp.sqrt(jnp.maximum(d2, 0.0))
    h = _K // 2
    da, db = dist[:, :h], dist[:, h:]
    mva = jnp.min(da, axis=1, keepdims=True)
    mvb = jnp.min(db, axis=1, keepdims=True)
    takeb = mvb < mva.astype(jnp.bfloat16).astype(jnp.float32)
    mv = jnp.where(takeb, mvb, mva)
    dw = jnp.where(takeb, db, da)
    iaf = iota_ref[...]
    idxf = jnp.min(jnp.where(dw == mv, iaf, jnp.float32(_K)), axis=1)
    idx = idxf.astype(jnp.int32) + jnp.where(takeb[:, 0], jnp.int32(h), 0)
    idx_ref[0, 0, :] = idx

    @pl.when(pl.program_id(0) == 0)
    def _():
        loss_ref[...] = jnp.zeros((1, 1), jnp.float32)

    loss_ref[...] += jnp.sum(mv * mv, axis=0, keepdims=True)

    @pl.when(pl.program_id(0) == _G - 1)
    def _():
        loss_ref[...] = ((1.0 + _COMMIT) / jnp.float32(_N * _D)) * loss_ref[...]


def _tc_argmin(flat_x, s1, emb_t, s2, iota_h):
    return pl.pallas_call(
        _argmin_body,
        grid=(_G,),
        in_specs=[
            pl.BlockSpec((_TM, _D), lambda i: (i, 0)),
            pl.BlockSpec((_TM, 1), lambda i: (i, 0)),
            pl.BlockSpec((_D, _K), lambda i: (0, 0)),
            pl.BlockSpec((1, _K), lambda i: (0, 0)),
            pl.BlockSpec((1, _K // 2), lambda i: (0, 0)),
        ],
        out_specs=[
            pl.BlockSpec((1, 1, _TM), lambda i: (i, 0, 0)),
            pl.BlockSpec((1, 1), lambda i: (0, 0)),
        ],
        out_shape=[
            jax.ShapeDtypeStruct((_G, 1, _TM), jnp.int32),
            jax.ShapeDtypeStruct((1, 1), jnp.float32),
        ],
    )(flat_x, s1, emb_t, s2, iota_h)


def _sc_gather(table, idx):
    info = plsc.get_sparse_core_info()
    nw = info.num_cores * info.num_subcores
    bpw = _N // nw
    mesh = plsc.VectorSubcoreMesh(core_axis_name="c", subcore_axis_name="s")

    @functools.partial(
        pl.kernel, mesh=mesh,
        compiler_params=pltpu.CompilerParams(use_tc_tiling_on_sc=False),
        out_type=jax.ShapeDtypeStruct((_N, _D), jnp.float32),
        scratch_types=[
            pltpu.VMEM((bpw,), jnp.int32),
            pltpu.VMEM((bpw, _D), jnp.float32),
            pltpu.SemaphoreType.DMA,
        ],
    )
    def k(table_hbm, idx_hbm, out_hbm, idx_v, rows_v, sem):
        wid = lax.axis_index("s") * info.num_cores + lax.axis_index("c")
        base = wid * bpw
        pltpu.sync_copy(idx_hbm.at[pl.ds(base, bpw)], idx_v)
        pltpu.async_copy(table_hbm.at[idx_v], rows_v, sem).wait()
        pltpu.sync_copy(rows_v, out_hbm.at[pl.ds(base, bpw)])

    return k(table, idx)


def kernel(x, embeddings):
    flat_x = x.reshape(_N, _D)
    s1 = jnp.sum(flat_x ** 2, axis=1, keepdims=True)
    s2 = jnp.sum(embeddings ** 2, axis=1)[None, :]
    iota_h = jnp.arange(_K // 2, dtype=jnp.float32)[None, :]
    idx3, loss_sum = _tc_argmin(flat_x, s1, embeddings.T, s2, iota_h)
    idx = idx3.reshape(_N)
    quantized = _sc_gather(embeddings, idx)
    return quantized.reshape(x.shape), loss_sum.reshape(())
